# baseline (speedup 1.0000x reference)
import functools

import jax
import jax.numpy as jnp
from jax import lax
from jax.experimental import pallas as pl
from jax.experimental.pallas import tpu as pltpu
from jax.experimental.pallas import tpu_sc as plsc

LABP = 16


def _proj_body(table_ref, w_ref, b_ref, out_ref):
    lab = w_ref.shape[1]
    pad = ((0, 0), (0, LABP - lab))
    w16 = jnp.pad(w_ref[...], pad)
    b16 = jnp.pad(b_ref[...], pad)
    out_ref[...] = (
        jnp.dot(table_ref[...], w16, preferred_element_type=jnp.float32) + b16
    )


def _make_gather(vp, lab, nl, nc):
    stp = (nl + 7) // 8
    mesh = plsc.VectorSubcoreMesh(core_axis_name="c", subcore_axis_name="s")

    @functools.partial(
        pl.kernel,
        mesh=mesh,
        out_type=jax.ShapeDtypeStruct((lab, stp, 32, 8, 128), jnp.float32),
        scratch_types=[
            pltpu.VMEM((vp * LABP,), jnp.float32),
            pltpu.VMEM((vp * (LABP + 1),), jnp.float32),
            pltpu.VMEM((128 * nl,), jnp.int32),
            pltpu.VMEM((lab, stp, 8, 128), jnp.float32),
        ],
        compiler_params=pltpu.CompilerParams(
            use_tc_tiling_on_sc=False,
            needs_layout_passes=False,
            disable_bounds_checks=True,
        ),
    )
    def gather_k(ptab_hbm, idx_hbm, out_hbm, ptab16_v, ptab_v, idx_v, comp_v):
        wid = lax.axis_index("s") * nc + lax.axis_index("c")
        n_per_w = 128 * nl
        pltpu.sync_copy(ptab_hbm, ptab16_v)
        pltpu.sync_copy(idx_hbm.at[pl.ds(wid * n_per_w, n_per_w)], idx_v)

        lane = lax.iota(jnp.int32, 16)
        lane_nl = lane * nl

        @plsc.parallel_loop(0, vp, 1, unroll=8)
        def restride(r):
            row = ptab16_v[pl.ds(r * LABP, 16)]
            plsc.store_scatter(ptab_v, [r * (LABP + 1) + lane], row)

        @plsc.parallel_loop(0, nl * 8, 1, unroll=16)
        def group(v):
            s = v >> 3
            b0 = (v & 7) * 16
            tok = plsc.load_gather(idx_v, [b0 * nl + s + lane_nl])
            a0 = tok * (LABP + 1)
            st = v >> 6
            si = (v >> 3) & 7
            for l in range(lab):
                comp_v[l, st, si, pl.ds(b0, 16)] = plsc.load_gather(
                    ptab_v, [a0 + l]
                )

        pltpu.sync_copy(comp_v, out_hbm.at[:, :, wid, :, :])

    return gather_k


def kernel(text, table, W, b):
    B, L = text.shape
    V, E = table.shape
    LAB = W.shape[1]

    proj = pl.pallas_call(
        _proj_body,
        out_shape=jax.ShapeDtypeStruct((V, LABP), jnp.float32),
    )(table, W, b.reshape(1, LAB))

    info = plsc.get_sparse_core_info()

    idx = text.reshape(B * L).astype(jnp.int32)
    y5 = _make_gather(V, LAB, L, info.num_cores)(proj.reshape(V * LABP), idx)
    y = y5.transpose(2, 4, 1, 3, 0).reshape(B, -1, LAB)
    return y[:, :L, :]


# speedup vs baseline: 1.0618x; 1.0618x over previous
import functools

import jax
import jax.numpy as jnp
from jax import lax
from jax.experimental import pallas as pl
from jax.experimental.pallas import tpu as pltpu
from jax.experimental.pallas import tpu_sc as plsc

LABP = 16


def _proj_body(table_ref, w_ref, b_ref, out_ref):
    lab = w_ref.shape[1]
    pad = ((0, 0), (0, LABP - lab))
    w16 = jnp.pad(w_ref[...], pad)
    b16 = jnp.pad(b_ref[...], pad)
    out_ref[...] = (
        jnp.dot(table_ref[...], w16, preferred_element_type=jnp.float32) + b16
    )


def _make_gather(vp, lab, nl, nc):
    stp = (nl + 7) // 8
    mesh = plsc.VectorSubcoreMesh(core_axis_name="c", subcore_axis_name="s")

    @functools.partial(
        pl.kernel,
        mesh=mesh,
        out_type=jax.ShapeDtypeStruct((lab, stp, 32, 8, 128), jnp.float32),
        scratch_types=[
            pltpu.VMEM((vp * LABP,), jnp.float32),
            pltpu.VMEM((vp * (LABP + 1),), jnp.float32),
            pltpu.VMEM((128 * nl,), jnp.int32),
            pltpu.VMEM((lab, stp, 8, 128), jnp.float32),
        ],
        compiler_params=pltpu.CompilerParams(
            use_tc_tiling_on_sc=False,
            needs_layout_passes=False,
            disable_bounds_checks=True,
        ),
    )
    def gather_k(ptab_hbm, idx_hbm, out_hbm, ptab16_v, ptab_v, idx_v, comp_v):
        wid = lax.axis_index("s") * nc + lax.axis_index("c")
        n_per_w = 128 * nl
        pltpu.sync_copy(ptab_hbm, ptab16_v)
        pltpu.sync_copy(idx_hbm.at[pl.ds(wid * n_per_w, n_per_w)], idx_v)

        lane = lax.iota(jnp.int32, 16)
        lane_nl = lane * nl

        @plsc.parallel_loop(0, vp, 1, unroll=8)
        def restride(r):
            row = ptab16_v[pl.ds(r * LABP, 16)]
            plsc.store_scatter(ptab_v, [r * (LABP + 1) + lane], row)

        @plsc.parallel_loop(0, nl * 8, 1, unroll=4)
        def group(v):
            s = v >> 3
            b0 = (v & 7) * 16
            tok = plsc.load_gather(idx_v, [b0 * nl + s + lane_nl])
            a0 = tok * (LABP + 1)
            st = v >> 6
            si = (v >> 3) & 7
            for l in range(lab):
                comp_v[l, st, si, pl.ds(b0, 16)] = plsc.load_gather(
                    ptab_v, [a0 + l]
                )

        pltpu.sync_copy(comp_v, out_hbm.at[:, :, wid, :, :])

    return gather_k


def kernel(text, table, W, b):
    B, L = text.shape
    V, E = table.shape
    LAB = W.shape[1]

    proj = pl.pallas_call(
        _proj_body,
        out_shape=jax.ShapeDtypeStruct((V, LABP), jnp.float32),
    )(table, W, b.reshape(1, LAB))

    info = plsc.get_sparse_core_info()

    idx = text.reshape(B * L).astype(jnp.int32)
    y5 = _make_gather(V, LAB, L, info.num_cores)(proj.reshape(V * LABP), idx)
    y = y5.transpose(2, 4, 1, 3, 0).reshape(B, -1, LAB)
    return y[:, :L, :]


# speedup vs baseline: 1.0635x; 1.0016x over previous
import functools

import jax
import jax.numpy as jnp
from jax import lax
from jax.experimental import pallas as pl
from jax.experimental.pallas import tpu as pltpu
from jax.experimental.pallas import tpu_sc as plsc

LABP = 16


def _proj_body(table_ref, w_ref, b_ref, out_ref):
    lab = w_ref.shape[1]
    pad = ((0, 0), (0, LABP - lab))
    w16 = jnp.pad(w_ref[...], pad)
    b16 = jnp.pad(b_ref[...], pad)
    out_ref[...] = (
        jnp.dot(table_ref[...], w16, preferred_element_type=jnp.float32) + b16
    )


def _make_gather(vp, lab, nl, nc):
    stp = (nl + 7) // 8
    mesh = plsc.VectorSubcoreMesh(core_axis_name="c", subcore_axis_name="s")

    @functools.partial(
        pl.kernel,
        mesh=mesh,
        out_type=jax.ShapeDtypeStruct((lab, stp, 32, 8, 128), jnp.float32),
        scratch_types=[
            pltpu.VMEM((vp * LABP,), jnp.float32),
            pltpu.VMEM((vp * (LABP + 1),), jnp.float32),
            pltpu.VMEM((128 * nl,), jnp.int32),
            pltpu.VMEM((lab, stp, 8, 128), jnp.float32),
        ],
        compiler_params=pltpu.CompilerParams(
            use_tc_tiling_on_sc=False,
            needs_layout_passes=False,
            disable_bounds_checks=True,
        ),
    )
    def gather_k(ptab_hbm, idx_hbm, out_hbm, ptab16_v, ptab_v, idx_v, comp_v):
        wid = lax.axis_index("s") * nc + lax.axis_index("c")
        n_per_w = 128 * nl
        pltpu.sync_copy(ptab_hbm, ptab16_v)
        pltpu.sync_copy(idx_hbm.at[pl.ds(wid * n_per_w, n_per_w)], idx_v)

        lane = lax.iota(jnp.int32, 16)
        lane_nl = lane * nl

        @plsc.parallel_loop(0, vp, 1, unroll=8)
        def restride(r):
            row = ptab16_v[pl.ds(r * LABP, 16)]
            plsc.store_scatter(ptab_v, [r * (LABP + 1) + lane], row)

        @plsc.parallel_loop(0, nl * 8, 1, unroll=2)
        def group(v):
            s = v >> 3
            b0 = (v & 7) * 16
            tok = plsc.load_gather(idx_v, [b0 * nl + s + lane_nl])
            a0 = tok * (LABP + 1)
            st = v >> 6
            si = (v >> 3) & 7
            for l in range(lab):
                comp_v[l, st, si, pl.ds(b0, 16)] = plsc.load_gather(
                    ptab_v, [a0 + l]
                )

        pltpu.sync_copy(comp_v, out_hbm.at[:, :, wid, :, :])

    return gather_k


def kernel(text, table, W, b):
    B, L = text.shape
    V, E = table.shape
    LAB = W.shape[1]

    proj = pl.pallas_call(
        _proj_body,
        out_shape=jax.ShapeDtypeStruct((V, LABP), jnp.float32),
    )(table, W, b.reshape(1, LAB))

    info = plsc.get_sparse_core_info()

    idx = text.reshape(B * L).astype(jnp.int32)
    y5 = _make_gather(V, LAB, L, info.num_cores)(proj.reshape(V * LABP), idx)
    y = y5.transpose(2, 4, 1, 3, 0).reshape(B, -1, LAB)
    return y[:, :L, :]


# speedup vs baseline: 1.1062x; 1.0402x over previous
import functools

import jax
import jax.numpy as jnp
from jax import lax
from jax.experimental import pallas as pl
from jax.experimental.pallas import tpu as pltpu
from jax.experimental.pallas import tpu_sc as plsc

LABP = 16


def _proj_body(table_ref, w_ref, b_ref, out_ref):
    lab = w_ref.shape[1]
    pad = ((0, 0), (0, LABP - lab))
    w16 = jnp.pad(w_ref[...], pad)
    b16 = jnp.pad(b_ref[...], pad)
    out_ref[...] = (
        jnp.dot(table_ref[...], w16, preferred_element_type=jnp.float32) + b16
    )


def _make_gather(vp, lab, nl, nc):
    stp = (nl + 7) // 8
    mesh = plsc.VectorSubcoreMesh(core_axis_name="c", subcore_axis_name="s")

    @functools.partial(
        pl.kernel,
        mesh=mesh,
        out_type=jax.ShapeDtypeStruct((lab, stp, 32, 8, 128), jnp.float32),
        scratch_types=[
            pltpu.VMEM((vp * LABP,), jnp.float32),
            pltpu.VMEM((vp * (LABP + 1),), jnp.float32),
            pltpu.VMEM((128 * nl,), jnp.int32),
            pltpu.VMEM((lab, stp, 8, 128), jnp.float32),
            pltpu.SemaphoreType.DMA,
        ],
        compiler_params=pltpu.CompilerParams(
            use_tc_tiling_on_sc=False,
            needs_layout_passes=False,
            disable_bounds_checks=True,
        ),
    )
    def gather_k(
        ptab_hbm, idx_hbm, out_hbm, ptab16_v, ptab_v, idx_v, comp_v, sem
    ):
        wid = lax.axis_index("s") * nc + lax.axis_index("c")
        n_per_w = 128 * nl
        idx_dma = pltpu.async_copy(
            idx_hbm.at[pl.ds(wid * n_per_w, n_per_w)], idx_v, sem
        )
        pltpu.sync_copy(ptab_hbm, ptab16_v)

        lane = lax.iota(jnp.int32, 16)
        lane_nl = lane * nl

        @plsc.parallel_loop(0, vp, 1, unroll=8)
        def restride(r):
            row = ptab16_v[pl.ds(r * LABP, 16)]
            plsc.store_scatter(ptab_v, [r * (LABP + 1) + lane], row)

        idx_dma.wait()

        def group(v):
            s = v >> 3
            b0 = (v & 7) * 16
            tok = plsc.load_gather(idx_v, [b0 * nl + s + lane_nl])
            a0 = tok * (LABP + 1)
            st = v >> 6
            si = (v >> 3) & 7
            for l in range(lab):
                comp_v[l, st, si, pl.ds(b0, 16)] = plsc.load_gather(
                    ptab_v, [a0 + l]
                )

        plsc.parallel_loop(0, 256, 1, unroll=2)(group)
        head_dma = pltpu.async_copy(
            comp_v.at[:, pl.ds(0, 4)],
            out_hbm.at[:, pl.ds(0, 4), wid, :, :],
            sem,
        )
        plsc.parallel_loop(256, nl * 8, 1, unroll=2)(group)
        head_dma.wait()
        pltpu.sync_copy(
            comp_v.at[:, pl.ds(4, stp - 4)],
            out_hbm.at[:, pl.ds(4, stp - 4), wid, :, :],
        )

    return gather_k


def kernel(text, table, W, b):
    B, L = text.shape
    V, E = table.shape
    LAB = W.shape[1]

    proj = pl.pallas_call(
        _proj_body,
        out_shape=jax.ShapeDtypeStruct((V, LABP), jnp.float32),
    )(table, W, b.reshape(1, LAB))

    info = plsc.get_sparse_core_info()

    idx = text.reshape(B * L).astype(jnp.int32)
    y5 = _make_gather(V, LAB, L, info.num_cores)(proj.reshape(V * LABP), idx)
    y = y5.transpose(2, 4, 1, 3, 0).reshape(B, -1, LAB)
    return y[:, :L, :]
